# Initial kernel scaffold; baseline (speedup 1.0000x reference)
import jax
import jax.numpy as jnp
from jax.experimental import pallas as pl


def kernel(species, positions, senders, receivers, embed, w_init, lin1_s, lin1_v, mlp_w1, mlp_b1, mlp_w2, mlp_b2, lin2_s, lin2_v, lin3_s, lin3_v):
    raise NotImplementedError("write your pallas kernel here")


# speedup vs baseline: 10.2747x; 10.2747x over previous
import math

import jax
import jax.numpy as jnp
import numpy as np
from jax.experimental import pallas as pl

_CUTOFF = 5.0


def _blk(n, pref):
    for b in (pref, 2048, 1024, 512, 256, 128, 64, 32, 16, 8):
        if n % b == 0:
            return b
    return n


def kernel(species, positions, senders, receivers, embed, w_init,
           lin1_s, lin1_v, mlp_w1, mlp_b1, mlp_w2, mlp_b2,
           lin2_s, lin2_v, lin3_s, lin3_v):
    N = species.shape[0]
    E = senders.shape[0]
    S = embed.shape[0]
    D = embed.shape[1]
    T, _, F = lin1_s.shape
    R = mlp_w1.shape[1]

    f32 = jnp.float32
    bn = _blk(N, 2000)
    be = _blk(E, 4000)
    gn = N // bn
    ge = E // be

    vals = np.linspace(0.0, _CUTOFF, R + 2)[1:-1]
    step = float(vals[1] - vals[0])
    vals2d = vals.reshape(1, R).astype(np.float32)
    inv_sqrt3 = float(1.0 / math.sqrt(3.0))
    sqrt3 = float(math.sqrt(3.0))

    def init_body(sp_ref, emb_ref, o_ref):
        sp = sp_ref[...]
        ids = jax.lax.broadcasted_iota(jnp.int32, (1, S), 1)
        oh = (sp == ids).astype(f32)
        o_ref[...] = oh @ emb_ref[...]

    def proj_body(xs_ref, xv_ref, ws_ref, wv_ref, o_ref):
        xs = xs_ref[...]
        wv = wv_ref[...]
        hs = xs @ ws_ref[...]
        hx = xv_ref[:, 0 * D:1 * D] @ wv
        hy = xv_ref[:, 1 * D:2 * D] @ wv
        hz = xv_ref[:, 2 * D:3 * D] @ wv
        o_ref[...] = jnp.concatenate([hs, hx, hy, hz], axis=1)

    def edge_body(ps_ref, pr_ref, h_ref, vals_ref, mw1_ref, mb1_ref,
                  mw2_ref, mb2_ref, o_ref):
        rx = pr_ref[:, 0:1] - ps_ref[:, 0:1]
        ry = pr_ref[:, 1:2] - ps_ref[:, 1:2]
        rz = pr_ref[:, 2:3] - ps_ref[:, 2:3]
        d = jnp.sqrt(rx * rx + ry * ry + rz * rz + 1e-12)
        diff = (d - vals_ref[...]) * (1.0 / step)
        fij = 1.12 * jnp.exp(-diff * diff)
        rcut = 0.5 * (jnp.cos(d * (math.pi / _CUTOFF)) + 1.0)
        rcut = rcut * (d < _CUTOFF).astype(f32)
        sx = rx * rcut
        sy = ry * rcut
        sz = rz * rcut
        dn = jnp.sqrt(sx * sx + sy * sy + sz * sz + 1e-12)
        inv = sqrt3 / dn
        yx = sx * inv
        yy = sy * inv
        yz = sz * inv
        w = jax.nn.silu(fij @ mw1_ref[...] + mb1_ref[...])
        w = (w @ mw2_ref[...] + mb2_ref[...]) * rcut
        a = h_ref[:, 0 * F:1 * F]
        bx = h_ref[:, 1 * F:2 * F]
        by = h_ref[:, 2 * F:3 * F]
        bz = h_ref[:, 3 * F:4 * F]
        t0b = (bx * yx + by * yy + bz * yz) * inv_sqrt3
        wa = w[:, 0 * F:1 * F]
        wb = w[:, 1 * F:2 * F]
        wc = w[:, 2 * F:3 * F]
        wd = w[:, 3 * F:4 * F]
        awc = a * wc
        o_ref[...] = jnp.concatenate(
            [a * wa, t0b * wb,
             awc * yx, bx * wd,
             awc * yy, by * wd,
             awc * yz, bz * wd], axis=1)

    def out_body(agg_ref, xs_ref, xv_ref, w2s_ref, w3s_ref, w2v_ref,
                 w3v_ref, xso_ref, xvo_ref):
        K = 2 * F
        sagg = agg_ref[:, 0 * K:1 * K]
        vx = agg_ref[:, 1 * K:2 * K]
        vy = agg_ref[:, 2 * K:3 * K]
        vz = agg_ref[:, 3 * K:4 * K]
        w2v = w2v_ref[...]
        w3v = w3v_ref[...]
        os_ = jax.nn.silu(sagg @ w2s_ref[...]) @ w3s_ref[...]
        ovx = (vx @ w2v) @ w3v
        ovy = (vy @ w2v) @ w3v
        ovz = (vz @ w2v) @ w3v
        xso_ref[...] = xs_ref[...] + os_
        xvo_ref[...] = jnp.concatenate(
            [xv_ref[:, 0 * D:1 * D] + ovx,
             xv_ref[:, 1 * D:2 * D] + ovy,
             xv_ref[:, 2 * D:3 * D] + ovz], axis=1)

    full = lambda shape: pl.BlockSpec(shape, lambda i: (0, 0))
    rows = lambda b, c: pl.BlockSpec((b, c), lambda i: (i, 0))

    init_call = pl.pallas_call(
        init_body,
        grid=(gn,),
        in_specs=[rows(bn, 1), full((S, D))],
        out_specs=rows(bn, D),
        out_shape=jax.ShapeDtypeStruct((N, D), f32),
    )

    proj_call = pl.pallas_call(
        proj_body,
        grid=(gn,),
        in_specs=[rows(bn, D), rows(bn, 3 * D), full((D, F)), full((D, F))],
        out_specs=rows(bn, 4 * F),
        out_shape=jax.ShapeDtypeStruct((N, 4 * F), f32),
    )

    edge_call = pl.pallas_call(
        edge_body,
        grid=(ge,),
        in_specs=[rows(be, 3), rows(be, 3), rows(be, 4 * F),
                  full((1, R)), full((R, F)), full((1, F)),
                  full((F, 4 * F)), full((1, 4 * F))],
        out_specs=rows(be, 8 * F),
        out_shape=jax.ShapeDtypeStruct((E, 8 * F), f32),
    )

    out_call = pl.pallas_call(
        out_body,
        grid=(gn,),
        in_specs=[rows(bn, 8 * F), rows(bn, D), rows(bn, 3 * D),
                  full((2 * F, D)), full((D, D)), full((2 * F, D)),
                  full((D, D))],
        out_specs=(rows(bn, D), rows(bn, 3 * D)),
        out_shape=(jax.ShapeDtypeStruct((N, D), f32),
                   jax.ShapeDtypeStruct((N, 3 * D), f32)),
    )

    embtab = (embed @ w_init) * (1.0 / math.sqrt(D))
    xs = init_call(species.reshape(N, 1).astype(jnp.int32), embtab)
    xv3 = jnp.zeros((N, 3 * D), f32)

    ps = jnp.take(positions, senders, axis=0)
    pr = jnp.take(positions, receivers, axis=0)

    inv_sqrt_d = 1.0 / math.sqrt(D)
    inv_sqrt_2f = 1.0 / math.sqrt(2 * F)
    for t in range(T):
        h = proj_call(xs, xv3, lin1_s[t] * inv_sqrt_d,
                      lin1_v[t] * inv_sqrt_d)
        hj = jnp.take(h, senders, axis=0)
        eo = edge_call(ps, pr, hj, jnp.asarray(vals2d),
                       mlp_w1[t], mlp_b1[t].reshape(1, F),
                       mlp_w2[t][:, :4 * F],
                       mlp_b2[t][:4 * F].reshape(1, 4 * F))
        agg = jax.ops.segment_sum(eo, receivers, num_segments=N)
        xs, xv3 = out_call(agg, xs, xv3,
                           lin2_s[t] * inv_sqrt_2f,
                           lin3_s[t] * inv_sqrt_d,
                           lin2_v[t] * inv_sqrt_2f,
                           lin3_v[t] * inv_sqrt_d)

    xv = xv3.reshape(N, 3, D).transpose(0, 2, 1).reshape(N, 3 * D)
    return jnp.concatenate([xs, xv], axis=-1)
